# baseline (speedup 1.0000x reference)
import functools

import jax
import jax.numpy as jnp
import numpy as np
from jax import lax
from jax.experimental import pallas as pl
from jax.experimental.pallas import tpu as pltpu
from jax.experimental.pallas import tpu_sc as plsc

N_ANGLE = 6
N_SIGMA = 4
N_ANG_PARTS = 6
N_RAD_PARTS = 3
N_PARTS = N_ANG_PARTS * N_RAD_PARTS
PIX = 4 * 24
SEG = N_SIGMA * N_PARTS * PIX
TOTAL = N_ANGLE * SEG
DESC = N_ANG_PARTS * N_RAD_PARTS * N_ANGLE

STATS_W = 160
OUT_W = 128

P = 8
GROUP = 16
N_WORKERS = 32


def _build_mmat() -> np.ndarray:
    c = np.arange(SEG)
    part = (c // PIX) % N_PARTS
    ap = part // N_RAD_PARTS
    m = np.zeros((SEG, 32), np.float32)
    m[c, part] = 1.0
    m[c, 18 + ap] = 1.0
    m[c, 24] = 1.0
    return m.astype(jnp.bfloat16)


_MMAT = _build_mmat()


def _tc_body(x_ref, m_ref, h_ref, v_ref):
    x = x_ref[...]
    xs = [x[:, a * SEG:(a + 1) * SEG] for a in range(N_ANGLE)]
    m = xs[0]
    for a in range(1, N_ANGLE):
        m = jnp.maximum(m, xs[a])
    prior = None
    inds = []
    for a in range(N_ANGLE):
        eq = xs[a] == m
        first = eq if prior is None else jnp.logical_and(eq, jnp.logical_not(prior))
        inds.append(first.astype(jnp.bfloat16))
        if a < N_ANGLE - 1:
            prior = eq if prior is None else jnp.logical_or(prior, eq)
    ind = jnp.concatenate(inds, axis=0)
    r = lax.dot_general(
        ind, m_ref[...],
        dimension_numbers=(((1,), (0,)), ((), ())),
        preferred_element_type=jnp.float32,
    )
    hs = [r[a * P:(a + 1) * P, 0:N_PARTS] for a in range(N_ANGLE)]
    ds = [r[a * P:(a + 1) * P, 18:24] for a in range(N_ANGLE)]
    ts = [r[a * P:(a + 1) * P, 24:25] for a in range(N_ANGLE)]
    h = jnp.concatenate(
        hs + ds + ts + [jnp.zeros((P, STATS_W - 150), jnp.float32)], axis=1
    )
    h_ref[...] = h
    v_ref[...] = jnp.sqrt(h * (1.0 / 6912.0))


def _tc_stats(x, mmat):
    bn = x.shape[0]
    return pl.pallas_call(
        _tc_body,
        grid=(bn // P,),
        in_specs=[
            pl.BlockSpec((P, TOTAL), lambda i: (i, 0)),
            pl.BlockSpec((SEG, 32), lambda i: (0, 0)),
        ],
        out_specs=[
            pl.BlockSpec((P, STATS_W), lambda i: (i, 0)),
            pl.BlockSpec((P, STATS_W), lambda i: (i, 0)),
        ],
        out_shape=[
            jax.ShapeDtypeStruct((bn, STATS_W), jnp.float32),
            jax.ShapeDtypeStruct((bn, STATS_W), jnp.float32),
        ],
    )(x, mmat)


def _full(val):
    return jnp.full((GROUP,), val, jnp.int32)


def _sc_group(h_v, v_v, o_v):
    lanes = lax.iota(jnp.int32, GROUP)

    def gat(ref, cols):
        return plsc.load_gather(ref, [lanes, cols])

    totals = [gat(h_v, _full(144 + a)) for a in range(N_ANGLE)]
    tmax = totals[0]
    for a in range(1, N_ANGLE):
        tmax = jnp.maximum(tmax, totals[a])
    pd = _full(0)
    for a in range(N_ANGLE - 1, -1, -1):
        pd = jnp.where(totals[a] == tmax, _full(a), pd)
    dis = [gat(h_v, pd * 6 + _full(108 + ap)) for ap in range(N_ANG_PARTS)]
    dmax = dis[0]
    for ap in range(1, N_ANG_PARTS):
        dmax = jnp.maximum(dmax, dis[ap])
    pap = _full(0)
    for ap in range(N_ANG_PARTS - 1, -1, -1):
        pap = jnp.where(dis[ap] == dmax, _full(ap), pap)
    for j in range(N_ANG_PARTS):
        sap = jnp.remainder(pap + _full(j), _full(N_ANG_PARTS))
        for k in range(N_RAD_PARTS):
            base_col = sap * 3 + _full(k)
            for l in range(N_ANGLE):
                sa = jnp.remainder(pd + _full(l), _full(N_ANGLE))
                val = gat(v_v, sa * 18 + base_col)
                plsc.store_scatter(
                    o_v, [lanes, _full(j * 18 + k * 6 + l)], val
                )


@functools.lru_cache(maxsize=2)
def _make_sc_kernel(bn):
    n_groups = bn // GROUP
    steps = -(-n_groups // N_WORKERS)
    mesh = plsc.VectorSubcoreMesh(core_axis_name="c", subcore_axis_name="s")

    @functools.partial(
        pl.kernel,
        mesh=mesh,
        compiler_params=pltpu.CompilerParams(
            use_tc_tiling_on_sc=False, needs_layout_passes=False
        ),
        out_type=jax.ShapeDtypeStruct((bn, OUT_W), jnp.float32),
        scratch_types=[
            pltpu.VMEM((GROUP, STATS_W), jnp.float32),
            pltpu.VMEM((GROUP, STATS_W), jnp.float32),
            pltpu.VMEM((GROUP, OUT_W), jnp.float32),
        ],
    )
    def sc_kernel(h_hbm, v_hbm, out_hbm, h_v, v_v, o_v):
        wid = lax.axis_index("s") * 2 + lax.axis_index("c")
        for gi in range(steps):
            g = gi * N_WORKERS + wid

            @pl.when(g < n_groups)
            def _():
                row0 = g * GROUP
                pltpu.sync_copy(h_hbm.at[pl.ds(row0, GROUP), :], h_v)
                pltpu.sync_copy(v_hbm.at[pl.ds(row0, GROUP), :], v_v)
                _sc_group(h_v, v_v, o_v)
                pltpu.sync_copy(o_v, out_hbm.at[pl.ds(row0, GROUP), :])

    return sc_kernel


def kernel(patches):
    b, n = patches.shape[0], patches.shape[1]
    bn = b * n
    x = patches.reshape(bn, TOTAL)
    h, v = _tc_stats(x, jnp.asarray(_MMAT))
    out = _make_sc_kernel(bn)(h, v)
    return out[:, :DESC].reshape(b, n, DESC)


# speedup vs baseline: 1.1887x; 1.1887x over previous
import functools

import jax
import jax.numpy as jnp
from jax import lax
from jax.experimental import pallas as pl
from jax.experimental.pallas import tpu as pltpu
from jax.experimental.pallas import tpu_sc as plsc

N_ANGLE = 6
N_SIGMA = 4
N_ANG_PARTS = 6
N_RAD_PARTS = 3
N_PARTS = N_ANG_PARTS * N_RAD_PARTS
PIX = 4 * 24
SEG = N_SIGMA * N_PARTS * PIX
TOTAL = N_ANGLE * SEG
DESC = N_ANG_PARTS * N_RAD_PARTS * N_ANGLE

STATS_W = 160
OUT_W = 128

P = 8
GROUP = 16
N_WORKERS = 32


def _tc_body(x_ref, h_ref, v_ref):
    x = x_ref[...]
    cnt = [None] * N_ANGLE
    for s in range(N_SIGMA):
        m = x[:, s]
        idx = jnp.zeros((P, N_PARTS, PIX), jnp.int32)
        for a in range(1, N_ANGLE):
            xa = x[:, a * N_SIGMA + s]
            gt = xa > m
            m = jnp.where(gt, xa, m)
            idx = jnp.where(gt, a, idx)
        for a in range(N_ANGLE):
            c = (idx == a).astype(jnp.float32)
            cnt[a] = c if cnt[a] is None else cnt[a] + c
    hs = [jnp.sum(cnt[a], axis=-1) for a in range(N_ANGLE)]
    ds = [
        jnp.concatenate(
            [
                jnp.sum(h[:, ap * 3:(ap + 1) * 3], axis=1, keepdims=True)
                for ap in range(N_ANG_PARTS)
            ],
            axis=1,
        )
        for h in hs
    ]
    ts = [jnp.sum(h, axis=1, keepdims=True) for h in hs]
    h = jnp.concatenate(
        hs + ds + ts + [jnp.zeros((P, STATS_W - 150), jnp.float32)], axis=1
    )
    h_ref[...] = h
    v_ref[...] = jnp.sqrt(h * (1.0 / 6912.0))


def _tc_stats(x):
    bn = x.shape[0]
    return pl.pallas_call(
        _tc_body,
        grid=(bn // P,),
        in_specs=[
            pl.BlockSpec((P, N_ANGLE * N_SIGMA, N_PARTS, PIX), lambda i: (i, 0, 0, 0)),
        ],
        out_specs=[
            pl.BlockSpec((P, STATS_W), lambda i: (i, 0)),
            pl.BlockSpec((P, STATS_W), lambda i: (i, 0)),
        ],
        out_shape=[
            jax.ShapeDtypeStruct((bn, STATS_W), jnp.float32),
            jax.ShapeDtypeStruct((bn, STATS_W), jnp.float32),
        ],
    )(x)


def _full(val):
    return jnp.full((GROUP,), val, jnp.int32)


def _sc_group(h_v, v_v, o_v):
    lanes = lax.iota(jnp.int32, GROUP)

    def gat(ref, cols):
        return plsc.load_gather(ref, [lanes, cols])

    totals = [gat(h_v, _full(144 + a)) for a in range(N_ANGLE)]
    tmax = totals[0]
    for a in range(1, N_ANGLE):
        tmax = jnp.maximum(tmax, totals[a])
    pd = _full(0)
    for a in range(N_ANGLE - 1, -1, -1):
        pd = jnp.where(totals[a] == tmax, _full(a), pd)
    dis = [gat(h_v, pd * 6 + _full(108 + ap)) for ap in range(N_ANG_PARTS)]
    dmax = dis[0]
    for ap in range(1, N_ANG_PARTS):
        dmax = jnp.maximum(dmax, dis[ap])
    pap = _full(0)
    for ap in range(N_ANG_PARTS - 1, -1, -1):
        pap = jnp.where(dis[ap] == dmax, _full(ap), pap)
    for j in range(N_ANG_PARTS):
        sap = jnp.remainder(pap + _full(j), _full(N_ANG_PARTS))
        for k in range(N_RAD_PARTS):
            base_col = sap * 3 + _full(k)
            for l in range(N_ANGLE):
                sa = jnp.remainder(pd + _full(l), _full(N_ANGLE))
                val = gat(v_v, sa * 18 + base_col)
                plsc.store_scatter(
                    o_v, [lanes, _full(j * 18 + k * 6 + l)], val
                )


@functools.lru_cache(maxsize=2)
def _make_sc_kernel(bn):
    n_groups = bn // GROUP
    steps = -(-n_groups // N_WORKERS)
    mesh = plsc.VectorSubcoreMesh(core_axis_name="c", subcore_axis_name="s")

    @functools.partial(
        pl.kernel,
        mesh=mesh,
        compiler_params=pltpu.CompilerParams(
            use_tc_tiling_on_sc=False, needs_layout_passes=False
        ),
        out_type=jax.ShapeDtypeStruct((bn, OUT_W), jnp.float32),
        scratch_types=[
            pltpu.VMEM((GROUP, STATS_W), jnp.float32),
            pltpu.VMEM((GROUP, STATS_W), jnp.float32),
            pltpu.VMEM((GROUP, OUT_W), jnp.float32),
        ],
    )
    def sc_kernel(h_hbm, v_hbm, out_hbm, h_v, v_v, o_v):
        wid = lax.axis_index("s") * 2 + lax.axis_index("c")
        for gi in range(steps):
            g = gi * N_WORKERS + wid

            @pl.when(g < n_groups)
            def _():
                row0 = g * GROUP
                pltpu.sync_copy(h_hbm.at[pl.ds(row0, GROUP), :], h_v)
                pltpu.sync_copy(v_hbm.at[pl.ds(row0, GROUP), :], v_v)
                _sc_group(h_v, v_v, o_v)
                pltpu.sync_copy(o_v, out_hbm.at[pl.ds(row0, GROUP), :])

    return sc_kernel


def kernel(patches):
    b, n = patches.shape[0], patches.shape[1]
    bn = b * n
    x = patches.reshape(bn, N_ANGLE * N_SIGMA, N_PARTS, PIX)
    h, v = _tc_stats(x)
    out = _make_sc_kernel(bn)(h, v)
    return out[:, :DESC].reshape(b, n, DESC)


# speedup vs baseline: 14.9505x; 12.5776x over previous
import functools

import jax
import jax.numpy as jnp
from jax import lax
from jax.experimental import pallas as pl
from jax.experimental.pallas import tpu as pltpu
from jax.experimental.pallas import tpu_sc as plsc

N_ANGLE = 6
N_SIGMA = 4
N_ANG_PARTS = 6
N_RAD_PARTS = 3
N_PARTS = N_ANG_PARTS * N_RAD_PARTS
PIX = 4 * 24
DESC = N_ANG_PARTS * N_RAD_PARTS * N_ANGLE

OUT_W = 128
GROUP = 16
N_WORKERS = 32


def _tc_body(x_ref, h_ref, v_ref):
    x = x_ref[...].reshape(N_ANGLE * N_SIGMA, PIX, x_ref.shape[-1])
    cnt = [None] * N_ANGLE
    for s in range(N_SIGMA):
        m = x[s]
        idx = jnp.zeros(m.shape, jnp.int32)
        for a in range(1, N_ANGLE):
            xa = x[a * N_SIGMA + s]
            gt = xa > m
            m = jnp.where(gt, xa, m)
            idx = jnp.where(gt, a, idx)
        for a in range(N_ANGLE):
            c = (idx == a).astype(jnp.float32)
            cnt[a] = c if cnt[a] is None else cnt[a] + c
    h = jnp.stack([jnp.sum(cnt[a], axis=0) for a in range(N_ANGLE)], axis=0)
    h = h.reshape(1, 1, N_ANGLE, x_ref.shape[-1])
    h_ref[...] = h
    v_ref[...] = jnp.sqrt(h * (1.0 / 6912.0))


def _tc_stats(xt):
    b, nf, npart, npix, n = xt.shape
    return pl.pallas_call(
        _tc_body,
        grid=(b, npart),
        in_specs=[
            pl.BlockSpec((1, nf, 1, npix, n), lambda i, q: (i, 0, q, 0, 0)),
        ],
        out_specs=[
            pl.BlockSpec((1, 1, N_ANGLE, n), lambda i, q: (i, q, 0, 0)),
            pl.BlockSpec((1, 1, N_ANGLE, n), lambda i, q: (i, q, 0, 0)),
        ],
        out_shape=[
            jax.ShapeDtypeStruct((b, npart, N_ANGLE, n), jnp.float32),
            jax.ShapeDtypeStruct((b, npart, N_ANGLE, n), jnp.float32),
        ],
    )(xt)


def _full(val):
    return jnp.full((GROUP,), val, jnp.int32)


def _sc_group(h_v, v_v, o_v):
    lanes = lax.iota(jnp.int32, GROUP)
    rows = [[h_v[q, a] for a in range(N_ANGLE)] for q in range(N_PARTS)]
    totals = []
    for a in range(N_ANGLE):
        t = rows[0][a]
        for q in range(1, N_PARTS):
            t = t + rows[q][a]
        totals.append(t)
    tmax = totals[0]
    for a in range(1, N_ANGLE):
        tmax = jnp.maximum(tmax, totals[a])
    pd = _full(0)
    for a in range(N_ANGLE - 1, -1, -1):
        pd = jnp.where(totals[a] == tmax, _full(a), pd)
    dis = []
    for ap in range(N_ANG_PARTS):
        d = None
        for rp in range(N_RAD_PARTS):
            g = plsc.load_gather(h_v, [_full(ap * 3 + rp), pd, lanes])
            d = g if d is None else d + g
        dis.append(d)
    dmax = dis[0]
    for ap in range(1, N_ANG_PARTS):
        dmax = jnp.maximum(dmax, dis[ap])
    pap = _full(0)
    for ap in range(N_ANG_PARTS - 1, -1, -1):
        pap = jnp.where(dis[ap] == dmax, _full(ap), pap)
    for j in range(N_ANG_PARTS):
        sap = jnp.remainder(pap + _full(j), _full(N_ANG_PARTS))
        for k in range(N_RAD_PARTS):
            qsrc = sap * 3 + _full(k)
            for l in range(N_ANGLE):
                sa = jnp.remainder(pd + _full(l), _full(N_ANGLE))
                val = plsc.load_gather(v_v, [qsrc, sa, lanes])
                plsc.store_scatter(
                    o_v, [lanes, _full(j * 18 + k * 6 + l)], val
                )


@functools.lru_cache(maxsize=2)
def _make_sc_kernel(b, n):
    groups_per_b = -(-n // GROUP)
    n_groups = b * groups_per_b
    steps = -(-n_groups // N_WORKERS)
    last_row = n - GROUP
    mesh = plsc.VectorSubcoreMesh(core_axis_name="c", subcore_axis_name="s")

    @functools.partial(
        pl.kernel,
        mesh=mesh,
        compiler_params=pltpu.CompilerParams(
            use_tc_tiling_on_sc=False, needs_layout_passes=False
        ),
        out_type=jax.ShapeDtypeStruct((b * n, OUT_W), jnp.float32),
        scratch_types=[
            pltpu.VMEM((N_PARTS, N_ANGLE, GROUP), jnp.float32),
            pltpu.VMEM((N_PARTS, N_ANGLE, GROUP), jnp.float32),
            pltpu.VMEM((GROUP, OUT_W), jnp.float32),
        ],
    )
    def sc_kernel(h_hbm, v_hbm, out_hbm, h_v, v_v, o_v):
        wid = lax.axis_index("s") * 2 + lax.axis_index("c")
        for gi in range(steps):
            g = gi * N_WORKERS + wid

            @pl.when(g < n_groups)
            def _():
                bb = g // groups_per_b
                gg = g % groups_per_b
                n0 = jnp.minimum(gg * GROUP, last_row)
                pltpu.sync_copy(h_hbm.at[bb, :, :, pl.ds(n0, GROUP)], h_v)
                pltpu.sync_copy(v_hbm.at[bb, :, :, pl.ds(n0, GROUP)], v_v)
                _sc_group(h_v, v_v, o_v)
                pltpu.sync_copy(o_v, out_hbm.at[pl.ds(bb * n + n0, GROUP), :])

    return sc_kernel


def kernel(patches):
    b, n = patches.shape[0], patches.shape[1]
    xt = jnp.transpose(patches, (0, 2, 3, 4, 1))
    h, v = _tc_stats(xt)
    out = _make_sc_kernel(b, n)(h, v)
    return out[:, :DESC].reshape(b, n, DESC)
